# baseline (speedup 1.0000x reference)
import jax
import jax.numpy as jnp
from jax import lax
from jax.experimental import pallas as pl
from jax.experimental.pallas import tpu as pltpu
from jax.experimental.pallas import tpu_sc as plsc

ALPHA = 1.0
GAMMA = 2.0
N_ROWS = 16384
N_CLS = 1000
BLK = 2048
NW = 32
BPW = N_ROWS // NW


def _sc_gather_body(x_hbm, t_hbm, out_hbm, t_v, g8_v, out_v, sem):
    c_id = lax.axis_index("c")
    s_id = lax.axis_index("s")
    wid = s_id * 2 + c_id
    base = wid * BPW
    pltpu.sync_copy(t_hbm.at[pl.ds(base, BPW)], t_v)
    for k in range(BPW // 128):
        pltpu.make_async_copy(
            x_hbm.at[t_v.at[pl.ds(k * 128, 128)], pl.ds(base + k * 128, 128)],
            g8_v.at[pl.ds(k * 128, 128)],
            sem,
        ).start()
    pltpu.make_async_copy(
        x_hbm.at[pl.ds(0, BPW), pl.ds(0, 128)], g8_v, sem
    ).wait()

    lanes = lax.broadcasted_iota(jnp.int32, (16,), 0)
    for g in range(BPW // 16):
        rows = g * 16 + lanes
        cols = rows & 127
        out_v[pl.ds(g * 16, 16)] = plsc.load_gather(g8_v, [rows, cols])
    pltpu.sync_copy(out_v, out_hbm.at[pl.ds(base, BPW)])


_sc_gather = pl.kernel(
    _sc_gather_body,
    out_type=jax.ShapeDtypeStruct((N_ROWS,), jnp.float32),
    mesh=plsc.VectorSubcoreMesh(core_axis_name="c", subcore_axis_name="s"),
    scratch_types=[
        pltpu.VMEM((BPW,), jnp.int32),
        pltpu.VMEM((BPW, 128), jnp.float32),
        pltpu.VMEM((BPW,), jnp.float32),
        pltpu.SemaphoreType.DMA,
    ],
    compiler_params=pltpu.CompilerParams(
        use_tc_tiling_on_sc=True, needs_layout_passes=False
    ),
)


def _stats_body(x_ref, out_ref):
    x = x_ref[...]
    m = jnp.max(x, axis=0, keepdims=True)
    s = jnp.sum(jnp.exp(x - m), axis=0, keepdims=True)
    out_ref[...] = jnp.concatenate([m, s], axis=0)


def _combine_body(ms_ref, xt_ref, out_ref):
    m = ms_ref[0:1, :]
    s = ms_ref[1:2, :]
    xt = xt_ref[...]
    logpt = (xt - m) - jnp.log(s)
    pt = jnp.exp(logpt)
    loss = -ALPHA * (1.0 - pt) * (1.0 - pt) * logpt
    out_ref[0, 0] = jnp.sum(loss) * (1.0 / N_ROWS)


def kernel(inputs, targets):
    xT = inputs.T
    t = targets.astype(jnp.int32)
    xt_vec = _sc_gather(xT, t)
    ms = pl.pallas_call(
        _stats_body,
        grid=(N_ROWS // BLK,),
        in_specs=[pl.BlockSpec((N_CLS, BLK), lambda i: (0, i))],
        out_specs=pl.BlockSpec((2, BLK), lambda i: (0, i)),
        out_shape=jax.ShapeDtypeStruct((2, N_ROWS), jnp.float32),
        compiler_params=pltpu.CompilerParams(
            dimension_semantics=("arbitrary",),
        ),
    )(xT)
    out = pl.pallas_call(
        _combine_body,
        in_specs=[
            pl.BlockSpec((2, N_ROWS), lambda: (0, 0)),
            pl.BlockSpec((1, N_ROWS), lambda: (0, 0)),
        ],
        out_specs=pl.BlockSpec((1, 1), memory_space=pltpu.SMEM),
        out_shape=jax.ShapeDtypeStruct((1, 1), jnp.float32),
    )(ms, xt_vec.reshape(1, N_ROWS))
    return out[0, 0]


# speedup vs baseline: 1.4784x; 1.4784x over previous
import jax
import jax.numpy as jnp
from jax import lax
from jax.experimental import pallas as pl
from jax.experimental.pallas import tpu as pltpu

ALPHA = 1.0
GAMMA = 2.0
N_ROWS = 16384
N_CLS = 1000
BLK = 2048


def _half_loss(x, t):
    m = jnp.max(x, axis=0, keepdims=True)
    s = jnp.sum(jnp.exp(x - m), axis=0, keepdims=True)
    cls = lax.broadcasted_iota(jnp.int32, (N_CLS, BLK), 0)
    onehot = cls == t
    xt = jnp.sum(jnp.where(onehot, x, 0.0), axis=0, keepdims=True)
    logpt = (xt - m) - jnp.log(s)
    pt = jnp.exp(logpt)
    loss = -ALPHA * (1.0 - pt) * (1.0 - pt) * logpt
    return jnp.sum(loss) * (1.0 / N_ROWS)


def _focal_body(xa_ref, xb_ref, ta_ref, tb_ref, out_ref):
    i = pl.program_id(0)
    bsum = _half_loss(xa_ref[...], ta_ref[...]) + _half_loss(
        xb_ref[...], tb_ref[...]
    )

    @pl.when(i == 0)
    def _():
        out_ref[0, 0] = 0.0

    out_ref[0, 0] += bsum


def kernel(inputs, targets):
    xt_view = inputs.T
    t2d = targets.astype(jnp.int32).reshape(1, N_ROWS)
    out = pl.pallas_call(
        _focal_body,
        grid=(N_ROWS // (2 * BLK),),
        in_specs=[
            pl.BlockSpec((N_CLS, BLK), lambda i: (0, i)),
            pl.BlockSpec((N_CLS, BLK), lambda i: (0, i + N_ROWS // (2 * BLK))),
            pl.BlockSpec((1, BLK), lambda i: (0, i)),
            pl.BlockSpec((1, BLK), lambda i: (0, i + N_ROWS // (2 * BLK))),
        ],
        out_specs=pl.BlockSpec(
            (1, 1), lambda i: (0, 0), memory_space=pltpu.SMEM
        ),
        out_shape=jax.ShapeDtypeStruct((1, 1), jnp.float32),
        compiler_params=pltpu.CompilerParams(
            dimension_semantics=("arbitrary",),
        ),
    )(xt_view, xt_view, t2d, t2d)
    return out[0, 0]


# speedup vs baseline: 1.6440x; 1.1121x over previous
import jax
import jax.numpy as jnp
from jax import lax
from jax.experimental import pallas as pl
from jax.experimental.pallas import tpu as pltpu

ALPHA = 1.0
GAMMA = 2.0
N_ROWS = 16384
N_CLS = 1000
BLK = 2048
CH = 8


def _focal_body(x_ref, t_ref, out_ref):
    i = pl.program_id(0)
    t = t_ref[...]
    sub_iota = lax.broadcasted_iota(jnp.int32, (CH, BLK), 0)
    m_acc = jnp.full((CH, BLK), -jnp.inf, jnp.float32)
    g_acc = jnp.zeros((CH, BLK), jnp.float32)
    for c in range(0, N_CLS, CH):
        xc = x_ref[c:c + CH, :]
        m_acc = jnp.maximum(m_acc, xc)
        g_acc = g_acc + jnp.where(sub_iota + c == t, xc, 0.0)
    m = jnp.max(m_acc, axis=0, keepdims=True)
    xt = jnp.sum(g_acc, axis=0, keepdims=True)

    s_acc = jnp.zeros((CH, BLK), jnp.float32)
    for c in range(0, N_CLS, CH):
        xc = x_ref[c:c + CH, :]
        s_acc = s_acc + jnp.exp(xc - m)
    s = jnp.sum(s_acc, axis=0, keepdims=True)

    logpt = (xt - m) - jnp.log(s)
    pt = jnp.exp(logpt)
    loss = -ALPHA * (1.0 - pt) * (1.0 - pt) * logpt
    bsum = jnp.sum(loss) * (1.0 / N_ROWS)

    @pl.when(i == 0)
    def _():
        out_ref[0, 0] = 0.0

    out_ref[0, 0] += bsum


def kernel(inputs, targets):
    xt_view = inputs.T
    t2d = targets.astype(jnp.int32).reshape(1, N_ROWS)
    out = pl.pallas_call(
        _focal_body,
        grid=(N_ROWS // BLK,),
        in_specs=[
            pl.BlockSpec((N_CLS, BLK), lambda i: (0, i)),
            pl.BlockSpec((1, BLK), lambda i: (0, i)),
        ],
        out_specs=pl.BlockSpec(
            (1, 1), lambda i: (0, 0), memory_space=pltpu.SMEM
        ),
        out_shape=jax.ShapeDtypeStruct((1, 1), jnp.float32),
        compiler_params=pltpu.CompilerParams(
            dimension_semantics=("arbitrary",),
        ),
    )(xt_view, t2d)
    return out[0, 0]
